# initial kernel scaffold (unmeasured)
import functools

import jax
import jax.numpy as jnp
from jax import lax
from jax.experimental import pallas as pl
from jax.experimental.pallas import tpu as pltpu

N_DEV = 8
B_PER = 2
SQ = 512
SKV = 512
H_PER = 8
DH = 64
D_MODEL = 768
D_SH = H_PER * DH
BLK = 64


def _body(x_ref, wq_ref, wo_ref, k_ref, v_ref, out_ref,
          wq_comm, wo_comm, wq_send, wq_recv, wo_send, wo_recv):
    my_pos = lax.axis_index("i")
    left = (my_pos - 1) % N_DEV
    right = (my_pos + 1) % N_DEV

    barrier_sem = pltpu.get_barrier_semaphore()
    for nbr in (left, right):
        pl.semaphore_signal(barrier_sem, inc=1, device_id=(nbr,),
                            device_id_type=pl.DeviceIdType.MESH)
    pl.semaphore_wait(barrier_sem, 2)

    rows = lax.broadcasted_iota(jnp.int32, (SQ, SKV), 0) // BLK
    cols = lax.broadcasted_iota(jnp.int32, (SQ, SKV), 1) // BLK
    mask = (cols <= rows)[None]

    x2d = x_ref[...].reshape(B_PER * SQ, D_MODEL)

    rdmas = []
    for h in range(N_DEV):
        if h < N_DEV - 1:
            src_wq = wq_ref if h == 0 else wq_comm.at[h]
            src_wo = wo_ref if h == 0 else wo_comm.at[h]
            rdma_wq = pltpu.make_async_remote_copy(
                src_ref=src_wq, dst_ref=wq_comm.at[h + 1],
                send_sem=wq_send.at[h], recv_sem=wq_recv.at[h + 1],
                device_id=(right,), device_id_type=pl.DeviceIdType.MESH)
            rdma_wo = pltpu.make_async_remote_copy(
                src_ref=src_wo, dst_ref=wo_comm.at[h + 1],
                send_sem=wo_send.at[h], recv_sem=wo_recv.at[h + 1],
                device_id=(right,), device_id_type=pl.DeviceIdType.MESH)
            rdma_wq.start()
            rdma_wo.start()
            rdmas.append((rdma_wq, rdma_wo))

        j = (my_pos - h) % N_DEV
        j8 = j * H_PER
        wq_cur = wq_ref[...] if h == 0 else wq_comm[h]
        wo_cur = wo_ref[...] if h == 0 else wo_comm[h]

        qf = jnp.dot(x2d, wq_cur,
                     preferred_element_type=jnp.float32).astype(jnp.bfloat16)
        for b in range(B_PER):
            qb = qf[b * SQ:(b + 1) * SQ].reshape(SQ, H_PER, DH)
            qh = qb.transpose(1, 0, 2)
            kb = k_ref[b, pl.ds(j8, H_PER)]
            vb = v_ref[b, pl.ds(j8, H_PER)]
            s = lax.dot_general(
                qh, kb, (((2,), (2,)), ((0,), (0,))),
                preferred_element_type=jnp.float32) * 0.125
            s = jnp.where(mask, s, -1e9)
            m = jnp.max(s, axis=-1, keepdims=True)
            e = jnp.exp(s - m)
            w = (e / jnp.sum(e, axis=-1, keepdims=True)).astype(jnp.bfloat16)
            ctx = lax.dot_general(
                w, vb, (((2,), (1,)), ((0,), (0,))),
                preferred_element_type=jnp.float32)
            cf = ctx.astype(jnp.bfloat16).transpose(1, 0, 2).reshape(SQ, D_SH)
            contrib = jnp.dot(cf, wo_cur,
                              preferred_element_type=jnp.float32)
            if h == 0:
                out_ref[b] = contrib
            else:
                out_ref[b] = out_ref[b] + contrib

        if h < N_DEV - 1:
            rdma_wq, rdma_wo = rdmas[h]
            rdma_wq.wait()
            rdma_wo.wait()


def kernel(x, Wq, K_ext, V_ext, Wo):
    p = lax.axis_index("i")

    ks = lax.dynamic_slice_in_dim(K_ext, p * B_PER, B_PER, axis=0)
    vs = lax.dynamic_slice_in_dim(V_ext, p * B_PER, B_PER, axis=0)
    kt = jnp.transpose(ks, (0, 2, 1, 3)).astype(jnp.bfloat16)
    vt = jnp.transpose(vs, (0, 2, 1, 3)).astype(jnp.bfloat16)
    xb = x.astype(jnp.bfloat16)
    wqb = Wq.astype(jnp.bfloat16)
    wob = Wo.astype(jnp.bfloat16)

    return pl.pallas_call(
        _body,
        out_shape=jax.ShapeDtypeStruct((B_PER, SQ, D_MODEL), jnp.float32),
        in_specs=[pl.BlockSpec(memory_space=pltpu.VMEM)] * 5,
        out_specs=pl.BlockSpec(memory_space=pltpu.VMEM),
        scratch_shapes=[
            pltpu.VMEM((N_DEV, D_MODEL, D_SH), jnp.bfloat16),
            pltpu.VMEM((N_DEV, D_SH, D_MODEL), jnp.bfloat16),
            pltpu.SemaphoreType.DMA((N_DEV,)),
            pltpu.SemaphoreType.DMA((N_DEV,)),
            pltpu.SemaphoreType.DMA((N_DEV,)),
            pltpu.SemaphoreType.DMA((N_DEV,)),
        ],
        compiler_params=pltpu.CompilerParams(collective_id=0),
    )(xb, wqb, kt, vt, wob)


# baseline (device time: 200221 ns/iter reference)
import functools

import jax
import jax.numpy as jnp
from jax import lax
from jax.experimental import pallas as pl
from jax.experimental.pallas import tpu as pltpu

N_DEV = 8
B_PER = 2
SQ = 512
SKV = 512
H_PER = 8
DH = 64
D_MODEL = 768
D_SH = H_PER * DH
BLK = 64


def _body(x_ref, wq_ref, k_ref, v_ref, wo_ref, out_ref,
          wq_comm, wo_comm, wq_send, wq_recv, wo_send, wo_recv):
    my_pos = lax.axis_index("i")
    left = (my_pos - 1) % N_DEV
    right = (my_pos + 1) % N_DEV

    barrier_sem = pltpu.get_barrier_semaphore()
    for nbr in (left, right):
        pl.semaphore_signal(barrier_sem, inc=1, device_id=(nbr,),
                            device_id_type=pl.DeviceIdType.MESH)
    pl.semaphore_wait(barrier_sem, 2)

    rows = lax.broadcasted_iota(jnp.int32, (SQ, SKV), 0) // BLK
    cols = lax.broadcasted_iota(jnp.int32, (SQ, SKV), 1) // BLK
    mask = (cols <= rows)[None]

    x2d = x_ref[...].reshape(B_PER * SQ, D_MODEL)

    rdmas = []
    for h in range(N_DEV):
        if h < N_DEV - 1:
            src_wq = wq_ref if h == 0 else wq_comm.at[h]
            src_wo = wo_ref if h == 0 else wo_comm.at[h]
            rdma_wq = pltpu.make_async_remote_copy(
                src_ref=src_wq, dst_ref=wq_comm.at[h + 1],
                send_sem=wq_send.at[h], recv_sem=wq_recv.at[h + 1],
                device_id=(right,), device_id_type=pl.DeviceIdType.MESH)
            rdma_wo = pltpu.make_async_remote_copy(
                src_ref=src_wo, dst_ref=wo_comm.at[h + 1],
                send_sem=wo_send.at[h], recv_sem=wo_recv.at[h + 1],
                device_id=(right,), device_id_type=pl.DeviceIdType.MESH)
            rdma_wq.start()
            rdma_wo.start()
            rdmas.append((rdma_wq, rdma_wo))

        j = (my_pos - h) % N_DEV
        j8 = j * H_PER
        wq_cur = wq_ref[...] if h == 0 else wq_comm[h]
        wo_cur = wo_ref[...] if h == 0 else wo_comm[h]

        qf = jnp.dot(x2d, wq_cur,
                     preferred_element_type=jnp.float32).astype(jnp.bfloat16)
        HC = 4
        for b in range(B_PER):
            qb = qf[b * SQ:(b + 1) * SQ].reshape(SQ, H_PER, DH)
            cfs = []
            for c in range(H_PER // HC):
                qh = qb[:, c * HC:(c + 1) * HC].transpose(1, 0, 2)
                kb = k_ref[b, pl.ds(j8 + c * HC, HC)]
                vb = v_ref[b, pl.ds(j8 + c * HC, HC)]
                s = lax.dot_general(
                    qh, kb, (((2,), (2,)), ((0,), (0,))),
                    preferred_element_type=jnp.float32) * 0.125
                s = jnp.where(mask, s, -1e9)
                m = jnp.max(s, axis=-1, keepdims=True)
                e = jnp.exp(s - m)
                w = (e / jnp.sum(e, axis=-1, keepdims=True)).astype(jnp.bfloat16)
                ctx = lax.dot_general(
                    w, vb, (((2,), (1,)), ((0,), (0,))),
                    preferred_element_type=jnp.float32)
                cfs.append(
                    ctx.astype(jnp.bfloat16).transpose(1, 0, 2).reshape(SQ, HC * DH))
            contrib = jnp.dot(jnp.concatenate(cfs, axis=1), wo_cur,
                              preferred_element_type=jnp.float32)
            if h == 0:
                out_ref[b] = contrib
            else:
                out_ref[b] = out_ref[b] + contrib

        if h < N_DEV - 1:
            rdma_wq, rdma_wo = rdmas[h]
            rdma_wq.wait()
            rdma_wo.wait()


def kernel(x, Wq, K_ext, V_ext, Wo):
    p = lax.axis_index("i")

    ks = lax.dynamic_slice_in_dim(K_ext, p * B_PER, B_PER, axis=0)
    vs = lax.dynamic_slice_in_dim(V_ext, p * B_PER, B_PER, axis=0)
    kt = jnp.transpose(ks, (0, 2, 1, 3)).astype(jnp.bfloat16)
    vt = jnp.transpose(vs, (0, 2, 1, 3)).astype(jnp.bfloat16)
    xb = x.astype(jnp.bfloat16)
    wqb = Wq.astype(jnp.bfloat16)
    wob = Wo.astype(jnp.bfloat16)

    return pl.pallas_call(
        _body,
        out_shape=jax.ShapeDtypeStruct((B_PER, SQ, D_MODEL), jnp.float32),
        in_specs=[pl.BlockSpec(memory_space=pltpu.VMEM)] * 5,
        out_specs=pl.BlockSpec(memory_space=pltpu.VMEM),
        scratch_shapes=[
            pltpu.VMEM((N_DEV, D_MODEL, D_SH), jnp.bfloat16),
            pltpu.VMEM((N_DEV, D_SH, D_MODEL), jnp.bfloat16),
            pltpu.SemaphoreType.DMA((N_DEV,)),
            pltpu.SemaphoreType.DMA((N_DEV,)),
            pltpu.SemaphoreType.DMA((N_DEV,)),
            pltpu.SemaphoreType.DMA((N_DEV,)),
        ],
        compiler_params=pltpu.CompilerParams(
            collective_id=0, vmem_limit_bytes=100 * 1024 * 1024),
    )(xb, wqb, kt, vt, wob)


# device time: 148929 ns/iter; 1.3444x vs baseline; 1.3444x over previous
import jax
import jax.numpy as jnp
from jax import lax
from jax.experimental import pallas as pl
from jax.experimental.pallas import tpu as pltpu

N_DEV = 8
B_PER = 2
SQ = 512
SKV = 512
H_PER = 8
HC = 4
DH = 64
D_MODEL = 768
D_HALF = HC * DH
BLK = 64


def _attn_group(x2d, k_ref, v_ref, out_ref, mask, wq_cur, wo_cur, head0,
                first):
    qf = jnp.dot(x2d, wq_cur,
                 preferred_element_type=jnp.float32).astype(jnp.bfloat16)
    for b in range(B_PER):
        qh = qf[b * SQ:(b + 1) * SQ].reshape(SQ, HC, DH).transpose(1, 0, 2)
        kb = k_ref[b, pl.ds(head0, HC)]
        vb = v_ref[b, pl.ds(head0, HC)]
        s = lax.dot_general(
            qh, kb, (((2,), (2,)), ((0,), (0,))),
            preferred_element_type=jnp.float32) * 0.125
        s = jnp.where(mask, s, -1e9)
        m = jnp.max(s, axis=-1, keepdims=True)
        e = jnp.exp(s - m)
        w = (e / jnp.sum(e, axis=-1, keepdims=True)).astype(jnp.bfloat16)
        ctx = lax.dot_general(
            w, vb, (((2,), (1,)), ((0,), (0,))),
            preferred_element_type=jnp.float32)
        cf = ctx.astype(jnp.bfloat16).transpose(1, 0, 2).reshape(SQ, D_HALF)
        contrib = jnp.dot(cf, wo_cur, preferred_element_type=jnp.float32)
        if first:
            out_ref[b] = contrib
        else:
            out_ref[b] = out_ref[b] + contrib


def _body(x_ref, wqa_ref, wqb_ref, woa_ref, wob_ref, k_ref, v_ref, out_ref,
          wqa_comm, woa_comm, wqb_comm, wob_comm,
          a_wq_send, a_wq_recv, a_wo_send, a_wo_recv,
          b_wq_send, b_wq_recv, b_wo_send, b_wo_recv):
    my_pos = lax.axis_index("i")
    left = (my_pos - 1) % N_DEV
    right = (my_pos + 1) % N_DEV

    barrier_sem = pltpu.get_barrier_semaphore()
    for nbr in (left, right):
        pl.semaphore_signal(barrier_sem, inc=1, device_id=(nbr,),
                            device_id_type=pl.DeviceIdType.MESH)
    pl.semaphore_wait(barrier_sem, 2)

    rows = lax.broadcasted_iota(jnp.int32, (SQ, SKV), 0) // BLK
    cols = lax.broadcasted_iota(jnp.int32, (SQ, SKV), 1) // BLK
    mask = (cols <= rows)[None]

    x2d = x_ref[...].reshape(B_PER * SQ, D_MODEL)

    rdmas = []
    for h in range(N_DEV):
        if h < N_DEV - 1:
            started = []
            for src0, comm_wq, comm_wo, s_wq, r_wq, s_wo, r_wo, tgt in (
                (wqa_ref, wqa_comm, woa_comm,
                 a_wq_send, a_wq_recv, a_wo_send, a_wo_recv, right),
                (wqb_ref, wqb_comm, wob_comm,
                 b_wq_send, b_wq_recv, b_wo_send, b_wo_recv, left),
            ):
                src_wq = src0 if h == 0 else comm_wq.at[h]
                src_wo = (woa_ref if tgt is right else wob_ref) \
                    if h == 0 else comm_wo.at[h]
                rdma_wq = pltpu.make_async_remote_copy(
                    src_ref=src_wq, dst_ref=comm_wq.at[h + 1],
                    send_sem=s_wq.at[h], recv_sem=r_wq.at[h + 1],
                    device_id=(tgt,), device_id_type=pl.DeviceIdType.MESH)
                rdma_wo = pltpu.make_async_remote_copy(
                    src_ref=src_wo, dst_ref=comm_wo.at[h + 1],
                    send_sem=s_wo.at[h], recv_sem=r_wo.at[h + 1],
                    device_id=(tgt,), device_id_type=pl.DeviceIdType.MESH)
                rdma_wq.start()
                rdma_wo.start()
                started.extend((rdma_wq, rdma_wo))
            rdmas.append(started)

        ja = (my_pos - h) % N_DEV
        jb = (my_pos + h) % N_DEV
        wqa_cur = wqa_ref[...] if h == 0 else wqa_comm[h]
        woa_cur = woa_ref[...] if h == 0 else woa_comm[h]
        wqb_cur = wqb_ref[...] if h == 0 else wqb_comm[h]
        wob_cur = wob_ref[...] if h == 0 else wob_comm[h]
        _attn_group(x2d, k_ref, v_ref, out_ref, mask,
                    wqa_cur, woa_cur, ja * H_PER, first=(h == 0))
        _attn_group(x2d, k_ref, v_ref, out_ref, mask,
                    wqb_cur, wob_cur, jb * H_PER + HC, first=False)

        if h < N_DEV - 1:
            for rdma in rdmas[h]:
                rdma.wait()


def kernel(x, Wq, K_ext, V_ext, Wo):
    p = lax.axis_index("i")

    ks = lax.dynamic_slice_in_dim(K_ext, p * B_PER, B_PER, axis=0)
    vs = lax.dynamic_slice_in_dim(V_ext, p * B_PER, B_PER, axis=0)
    kt = jnp.transpose(ks, (0, 2, 1, 3)).astype(jnp.bfloat16)
    vt = jnp.transpose(vs, (0, 2, 1, 3)).astype(jnp.bfloat16)
    xb = x.astype(jnp.bfloat16)
    wqb16 = Wq.astype(jnp.bfloat16)
    wob16 = Wo.astype(jnp.bfloat16)
    wqa, wqb = wqb16[:, :D_HALF], wqb16[:, D_HALF:]
    woa, wob = wob16[:D_HALF], wob16[D_HALF:]

    return pl.pallas_call(
        _body,
        out_shape=jax.ShapeDtypeStruct((B_PER, SQ, D_MODEL), jnp.float32),
        in_specs=[pl.BlockSpec(memory_space=pltpu.VMEM)] * 7,
        out_specs=pl.BlockSpec(memory_space=pltpu.VMEM),
        scratch_shapes=[
            pltpu.VMEM((N_DEV, D_MODEL, D_HALF), jnp.bfloat16),
            pltpu.VMEM((N_DEV, D_HALF, D_MODEL), jnp.bfloat16),
            pltpu.VMEM((N_DEV, D_MODEL, D_HALF), jnp.bfloat16),
            pltpu.VMEM((N_DEV, D_HALF, D_MODEL), jnp.bfloat16),
        ] + [pltpu.SemaphoreType.DMA((N_DEV,))] * 8,
        compiler_params=pltpu.CompilerParams(
            collective_id=0, vmem_limit_bytes=100 * 1024 * 1024),
    )(xb, wqa, wqb, woa, wob, kt, vt)


# device time: 130351 ns/iter; 1.5360x vs baseline; 1.1425x over previous
import jax
import jax.numpy as jnp
from jax import lax
from jax.experimental import pallas as pl
from jax.experimental.pallas import tpu as pltpu

N_DEV = 8
B_PER = 2
SQ = 512
SKV = 512
H_PER = 8
HC = 4
DH = 64
D_MODEL = 768
D_HALF = HC * DH
BLK = 64


def _attn_group(x2d, k_ref, v_ref, cf_ref, mask, wq_cur, head0, h):
    qf = jnp.dot(x2d, wq_cur,
                 preferred_element_type=jnp.float32).astype(jnp.bfloat16)
    for b in range(B_PER):
        qh = qf[b * SQ:(b + 1) * SQ].reshape(SQ, HC, DH).transpose(1, 0, 2)
        kb = k_ref[b, pl.ds(head0, HC)]
        vb = v_ref[b, pl.ds(head0, HC)]
        s = lax.dot_general(
            qh, kb, (((2,), (1,)), ((0,), (0,))),
            preferred_element_type=jnp.float32)
        e = jnp.where(mask, jnp.exp(s), 0.0)
        denom = jnp.sum(e, axis=-1)
        ctx_t = lax.dot_general(
            vb, e.astype(jnp.bfloat16), (((2,), (2,)), ((0,), (0,))),
            preferred_element_type=jnp.float32)
        ctx_t = ctx_t / denom[:, None, :]
        cf_ref[pl.ds(b * SQ, SQ), pl.ds(h * D_HALF, D_HALF)] = (
            ctx_t.astype(jnp.bfloat16).transpose(2, 0, 1).reshape(SQ, D_HALF))


def _body(x_ref, wqa_ref, wqb_ref, woa_ref, wob_ref, k_ref, v_ref, out_ref,
          wqa_comm, woa_comm, wqb_comm, wob_comm, cfa_ref, cfb_ref,
          a_wq_send, a_wq_recv, a_wo_send, a_wo_recv,
          b_wq_send, b_wq_recv, b_wo_send, b_wo_recv):
    my_pos = lax.axis_index("i")
    left = (my_pos - 1) % N_DEV
    right = (my_pos + 1) % N_DEV

    barrier_sem = pltpu.get_barrier_semaphore()
    for nbr in (left, right):
        pl.semaphore_signal(barrier_sem, inc=1, device_id=(nbr,),
                            device_id_type=pl.DeviceIdType.MESH)
    pl.semaphore_wait(barrier_sem, 2)

    rows = lax.broadcasted_iota(jnp.int32, (SQ, SKV), 0) // BLK
    cols = lax.broadcasted_iota(jnp.int32, (SQ, SKV), 1) // BLK
    mask = (cols <= rows)[None]

    x2d = x_ref[...].reshape(B_PER * SQ, D_MODEL)

    woa_comm[0] = woa_ref[...]
    wob_comm[0] = wob_ref[...]

    rdmas = []
    for h in range(N_DEV):
        if h < N_DEV - 1:
            started = []
            for wq_src0, comm_wq, comm_wo, s_wq, r_wq, s_wo, r_wo, tgt in (
                (wqa_ref, wqa_comm, woa_comm,
                 a_wq_send, a_wq_recv, a_wo_send, a_wo_recv, right),
                (wqb_ref, wqb_comm, wob_comm,
                 b_wq_send, b_wq_recv, b_wo_send, b_wo_recv, left),
            ):
                src_wq = wq_src0 if h == 0 else comm_wq.at[h]
                rdma_wq = pltpu.make_async_remote_copy(
                    src_ref=src_wq, dst_ref=comm_wq.at[h + 1],
                    send_sem=s_wq.at[h], recv_sem=r_wq.at[h + 1],
                    device_id=(tgt,), device_id_type=pl.DeviceIdType.MESH)
                rdma_wo = pltpu.make_async_remote_copy(
                    src_ref=comm_wo.at[h], dst_ref=comm_wo.at[h + 1],
                    send_sem=s_wo.at[h], recv_sem=r_wo.at[h + 1],
                    device_id=(tgt,), device_id_type=pl.DeviceIdType.MESH)
                rdma_wq.start()
                rdma_wo.start()
                started.extend((rdma_wq, rdma_wo))
            rdmas.append(started)

        ja = (my_pos - h) % N_DEV
        jb = (my_pos + h) % N_DEV
        wqa_cur = wqa_ref[...] if h == 0 else wqa_comm[h]
        wqb_cur = wqb_ref[...] if h == 0 else wqb_comm[h]
        _attn_group(x2d, k_ref, v_ref, cfa_ref, mask, wqa_cur,
                    ja * H_PER, h)
        _attn_group(x2d, k_ref, v_ref, cfb_ref, mask, wqb_cur,
                    jb * H_PER + HC, h)

        if h < N_DEV - 1:
            for rdma in rdmas[h]:
                rdma.wait()

    woa_all = woa_comm[...].reshape(N_DEV * D_HALF, D_MODEL)
    wob_all = wob_comm[...].reshape(N_DEV * D_HALF, D_MODEL)
    out = jnp.dot(cfa_ref[...], woa_all,
                  preferred_element_type=jnp.float32)
    out = out + jnp.dot(cfb_ref[...], wob_all,
                        preferred_element_type=jnp.float32)
    out_ref[...] = out.reshape(B_PER, SQ, D_MODEL)


def kernel(x, Wq, K_ext, V_ext, Wo):
    p = lax.axis_index("i")

    ks = lax.dynamic_slice_in_dim(K_ext, p * B_PER, B_PER, axis=0)
    vs = lax.dynamic_slice_in_dim(V_ext, p * B_PER, B_PER, axis=0)
    kt = jnp.transpose(ks, (0, 2, 3, 1)).astype(jnp.bfloat16)
    vt = jnp.transpose(vs, (0, 2, 3, 1)).astype(jnp.bfloat16)
    xb = x.astype(jnp.bfloat16)
    wq16 = (Wq * 0.125).astype(jnp.bfloat16)
    wo16 = Wo.astype(jnp.bfloat16)
    wqa, wqb = wq16[:, :D_HALF], wq16[:, D_HALF:]
    woa, wob = wo16[:D_HALF], wo16[D_HALF:]

    return pl.pallas_call(
        _body,
        out_shape=jax.ShapeDtypeStruct((B_PER, SQ, D_MODEL), jnp.float32),
        in_specs=[pl.BlockSpec(memory_space=pltpu.VMEM)] * 7,
        out_specs=pl.BlockSpec(memory_space=pltpu.VMEM),
        scratch_shapes=[
            pltpu.VMEM((N_DEV, D_MODEL, D_HALF), jnp.bfloat16),
            pltpu.VMEM((N_DEV, D_HALF, D_MODEL), jnp.bfloat16),
            pltpu.VMEM((N_DEV, D_MODEL, D_HALF), jnp.bfloat16),
            pltpu.VMEM((N_DEV, D_HALF, D_MODEL), jnp.bfloat16),
            pltpu.VMEM((B_PER * SQ, N_DEV * D_HALF), jnp.bfloat16),
            pltpu.VMEM((B_PER * SQ, N_DEV * D_HALF), jnp.bfloat16),
        ] + [pltpu.SemaphoreType.DMA((N_DEV,))] * 8,
        compiler_params=pltpu.CompilerParams(
            collective_id=0, vmem_limit_bytes=100 * 1024 * 1024),
    )(xb, wqa, wqb, woa, wob, kt, vt)


# device time: 129903 ns/iter; 1.5413x vs baseline; 1.0034x over previous
import jax
import jax.numpy as jnp
from jax import lax
from jax.experimental import pallas as pl
from jax.experimental.pallas import tpu as pltpu

N_DEV = 8
B_PER = 2
SQ = 512
SKV = 512
H_PER = 8
HC = 4
DH = 64
D_MODEL = 768
D_HALF = HC * DH
BLK = 64


def _attn_group(x2d, k_ref, v_ref, cf_ref, mask, wq_cur, head0, h):
    qf = jnp.dot(x2d, wq_cur,
                 preferred_element_type=jnp.float32).astype(jnp.bfloat16)
    HQ = SQ // 2
    for b in range(B_PER):
        qh = qf[b * SQ:(b + 1) * SQ].reshape(SQ, HC, DH).transpose(1, 0, 2)
        kb = k_ref[b, pl.ds(head0, HC)]
        vb = v_ref[b, pl.ds(head0, HC)]
        ctx_h, den_h = [], []
        for lo, klen in ((0, HQ), (HQ, SKV)):
            s = lax.dot_general(
                qh[:, lo:lo + HQ], kb[:, :, :klen],
                (((2,), (1,)), ((0,), (0,))),
                preferred_element_type=jnp.float32)
            e = jnp.where(mask[:, lo:lo + HQ, :klen],
                          jnp.exp(s.astype(jnp.bfloat16)),
                          jnp.bfloat16(0.0))
            den_h.append(jnp.sum(e, axis=-1, dtype=jnp.float32))
            ctx_h.append(lax.dot_general(
                vb[:, :, :klen], e, (((2,), (2,)), ((0,), (0,))),
                preferred_element_type=jnp.float32))
        ctx_t = jnp.concatenate(ctx_h, axis=2)
        denom = jnp.concatenate(den_h, axis=1)
        ctx_t = ctx_t / denom[:, None, :]
        cf_ref[pl.ds(b * SQ, SQ), pl.ds(h * D_HALF, D_HALF)] = (
            ctx_t.astype(jnp.bfloat16).transpose(2, 0, 1).reshape(SQ, D_HALF))


def _body(x_ref, wqa_ref, wqb_ref, woa_ref, wob_ref, k_ref, v_ref, out_ref,
          wqa_comm, woa_comm, wqb_comm, wob_comm, cfa_ref, cfb_ref,
          a_wq_send, a_wq_recv, a_wo_send, a_wo_recv,
          b_wq_send, b_wq_recv, b_wo_send, b_wo_recv):
    my_pos = lax.axis_index("i")
    left = (my_pos - 1) % N_DEV
    right = (my_pos + 1) % N_DEV

    barrier_sem = pltpu.get_barrier_semaphore()
    for nbr in (left, right):
        pl.semaphore_signal(barrier_sem, inc=1, device_id=(nbr,),
                            device_id_type=pl.DeviceIdType.MESH)
    pl.semaphore_wait(barrier_sem, 2)

    rows = lax.broadcasted_iota(jnp.int32, (SQ, SKV), 0) // BLK
    cols = lax.broadcasted_iota(jnp.int32, (SQ, SKV), 1) // BLK
    mask = (cols <= rows)[None]

    x2d = x_ref[...].reshape(B_PER * SQ, D_MODEL)

    woa_comm[0] = woa_ref[...]
    wob_comm[0] = wob_ref[...]

    rdmas = []
    for h in range(N_DEV):
        if h < N_DEV - 1:
            started = []
            for wq_src0, comm_wq, comm_wo, s_wq, r_wq, s_wo, r_wo, tgt in (
                (wqa_ref, wqa_comm, woa_comm,
                 a_wq_send, a_wq_recv, a_wo_send, a_wo_recv, right),
                (wqb_ref, wqb_comm, wob_comm,
                 b_wq_send, b_wq_recv, b_wo_send, b_wo_recv, left),
            ):
                src_wq = wq_src0 if h == 0 else comm_wq.at[h]
                rdma_wq = pltpu.make_async_remote_copy(
                    src_ref=src_wq, dst_ref=comm_wq.at[h + 1],
                    send_sem=s_wq.at[h], recv_sem=r_wq.at[h + 1],
                    device_id=(tgt,), device_id_type=pl.DeviceIdType.MESH)
                rdma_wo = pltpu.make_async_remote_copy(
                    src_ref=comm_wo.at[h], dst_ref=comm_wo.at[h + 1],
                    send_sem=s_wo.at[h], recv_sem=r_wo.at[h + 1],
                    device_id=(tgt,), device_id_type=pl.DeviceIdType.MESH)
                rdma_wq.start()
                rdma_wo.start()
                started.extend((rdma_wq, rdma_wo))
            rdmas.append(started)

        ja = (my_pos - h) % N_DEV
        jb = (my_pos + h) % N_DEV
        wqa_cur = wqa_ref[...] if h == 0 else wqa_comm[h]
        wqb_cur = wqb_ref[...] if h == 0 else wqb_comm[h]
        _attn_group(x2d, k_ref, v_ref, cfa_ref, mask, wqa_cur,
                    ja * H_PER, h)
        _attn_group(x2d, k_ref, v_ref, cfb_ref, mask, wqb_cur,
                    jb * H_PER + HC, h)

        if h < N_DEV - 1:
            for rdma in rdmas[h]:
                rdma.wait()

    woa_all = woa_comm[...].reshape(N_DEV * D_HALF, D_MODEL)
    wob_all = wob_comm[...].reshape(N_DEV * D_HALF, D_MODEL)
    out = jnp.dot(cfa_ref[...], woa_all,
                  preferred_element_type=jnp.float32)
    out = out + jnp.dot(cfb_ref[...], wob_all,
                        preferred_element_type=jnp.float32)
    out_ref[...] = out.reshape(B_PER, SQ, D_MODEL)


def kernel(x, Wq, K_ext, V_ext, Wo):
    p = lax.axis_index("i")

    ks = lax.dynamic_slice_in_dim(K_ext, p * B_PER, B_PER, axis=0)
    vs = lax.dynamic_slice_in_dim(V_ext, p * B_PER, B_PER, axis=0)
    kt = jnp.transpose(ks, (0, 2, 3, 1)).astype(jnp.bfloat16)
    vt = jnp.transpose(vs, (0, 2, 3, 1)).astype(jnp.bfloat16)
    xb = x.astype(jnp.bfloat16)
    wq16 = (Wq * 0.125).astype(jnp.bfloat16)
    wo16 = Wo.astype(jnp.bfloat16)
    wqa, wqb = wq16[:, :D_HALF], wq16[:, D_HALF:]
    woa, wob = wo16[:D_HALF], wo16[D_HALF:]

    return pl.pallas_call(
        _body,
        out_shape=jax.ShapeDtypeStruct((B_PER, SQ, D_MODEL), jnp.float32),
        in_specs=[pl.BlockSpec(memory_space=pltpu.VMEM)] * 7,
        out_specs=pl.BlockSpec(memory_space=pltpu.VMEM),
        scratch_shapes=[
            pltpu.VMEM((N_DEV, D_MODEL, D_HALF), jnp.bfloat16),
            pltpu.VMEM((N_DEV, D_HALF, D_MODEL), jnp.bfloat16),
            pltpu.VMEM((N_DEV, D_MODEL, D_HALF), jnp.bfloat16),
            pltpu.VMEM((N_DEV, D_HALF, D_MODEL), jnp.bfloat16),
            pltpu.VMEM((B_PER * SQ, N_DEV * D_HALF), jnp.bfloat16),
            pltpu.VMEM((B_PER * SQ, N_DEV * D_HALF), jnp.bfloat16),
        ] + [pltpu.SemaphoreType.DMA((N_DEV,))] * 8,
        compiler_params=pltpu.CompilerParams(
            collective_id=0, vmem_limit_bytes=100 * 1024 * 1024),
    )(xb, wqa, wqb, woa, wob, kt, vt)


# device time: 127480 ns/iter; 1.5706x vs baseline; 1.0190x over previous
import jax
import jax.numpy as jnp
from jax import lax
from jax.experimental import pallas as pl
from jax.experimental.pallas import tpu as pltpu

N_DEV = 8
B_PER = 2
SQ = 512
SKV = 512
H_PER = 8
HC = 4
DH = 64
D_MODEL = 768
D_HALF = HC * DH
BLK = 64


def _attn_group(x2d, k_ref, v_ref, cf_ref, mask, wq_cur, head0, h):
    qf = jnp.dot(x2d, wq_cur,
                 preferred_element_type=jnp.float32).astype(jnp.bfloat16)
    HQ = SQ // 2
    for b in range(B_PER):
        qh = qf[b * SQ:(b + 1) * SQ].reshape(SQ, HC, DH).transpose(1, 0, 2)
        kb = k_ref[b, pl.ds(head0, HC)]
        vb = v_ref[b, pl.ds(head0, HC)]
        ctx_h, den_h = [], []
        for lo, klen in ((0, HQ), (HQ, SKV)):
            s = lax.dot_general(
                qh[:, lo:lo + HQ], kb[:, :, :klen],
                (((2,), (1,)), ((0,), (0,))),
                preferred_element_type=jnp.float32)
            e = jnp.where(mask[:, lo:lo + HQ, :klen],
                          jnp.exp(s.astype(jnp.bfloat16)),
                          jnp.bfloat16(0.0))
            den_h.append(jnp.sum(e, axis=-1, dtype=jnp.float32))
            ctx_h.append(lax.dot_general(
                vb[:, :, :klen], e, (((2,), (2,)), ((0,), (0,))),
                preferred_element_type=jnp.float32))
        ctx_t = jnp.concatenate(ctx_h, axis=2)
        denom = jnp.concatenate(den_h, axis=1)
        ctx_t = ctx_t / denom[:, None, :]
        cf_ref[pl.ds(b * SQ, SQ), pl.ds(h * D_HALF, D_HALF)] = (
            ctx_t.astype(jnp.bfloat16).transpose(2, 0, 1).reshape(SQ, D_HALF))


_SUCC = (1, 2, 3, 7, 0, 4, 5, 6)
_PRED = (4, 0, 1, 2, 5, 6, 7, 3)


def _perm(table, p):
    out = jnp.int32(table[0])
    for k in range(1, N_DEV):
        out = jnp.where(p == k, jnp.int32(table[k]), out)
    return out


def _body(x_ref, wqa_ref, wqb_ref, woa_ref, wob_ref, k_ref, v_ref, out_ref,
          wqa_comm, woa_comm, wqb_comm, wob_comm, cfa_ref, cfb_ref,
          a_wq_send, a_wq_recv, a_wo_send, a_wo_recv,
          b_wq_send, b_wq_recv, b_wo_send, b_wo_recv):
    my_pos = lax.axis_index("i")
    left = _perm(_PRED, my_pos)
    right = _perm(_SUCC, my_pos)

    barrier_sem = pltpu.get_barrier_semaphore()
    for nbr in (left, right):
        pl.semaphore_signal(barrier_sem, inc=1, device_id=(nbr,),
                            device_id_type=pl.DeviceIdType.MESH)
    pl.semaphore_wait(barrier_sem, 2)

    rows = lax.broadcasted_iota(jnp.int32, (SQ, SKV), 0) // BLK
    cols = lax.broadcasted_iota(jnp.int32, (SQ, SKV), 1) // BLK
    mask = (cols <= rows)[None]

    x2d = x_ref[...].reshape(B_PER * SQ, D_MODEL)

    woa_comm[0] = woa_ref[...]
    wob_comm[0] = wob_ref[...]

    rdmas = []
    ja = my_pos
    jb = my_pos
    for h in range(N_DEV):
        if h > 0:
            ja = _perm(_PRED, ja)
            jb = _perm(_SUCC, jb)
        if h < N_DEV - 1:
            started = []
            for wq_src0, comm_wq, comm_wo, s_wq, r_wq, s_wo, r_wo, tgt in (
                (wqa_ref, wqa_comm, woa_comm,
                 a_wq_send, a_wq_recv, a_wo_send, a_wo_recv, right),
                (wqb_ref, wqb_comm, wob_comm,
                 b_wq_send, b_wq_recv, b_wo_send, b_wo_recv, left),
            ):
                src_wq = wq_src0 if h == 0 else comm_wq.at[h]
                rdma_wq = pltpu.make_async_remote_copy(
                    src_ref=src_wq, dst_ref=comm_wq.at[h + 1],
                    send_sem=s_wq.at[h], recv_sem=r_wq.at[h + 1],
                    device_id=(tgt,), device_id_type=pl.DeviceIdType.MESH)
                rdma_wo = pltpu.make_async_remote_copy(
                    src_ref=comm_wo.at[h], dst_ref=comm_wo.at[h + 1],
                    send_sem=s_wo.at[h], recv_sem=r_wo.at[h + 1],
                    device_id=(tgt,), device_id_type=pl.DeviceIdType.MESH)
                rdma_wq.start()
                rdma_wo.start()
                started.extend((rdma_wq, rdma_wo))
            rdmas.append(started)

        wqa_cur = wqa_ref[...] if h == 0 else wqa_comm[h]
        wqb_cur = wqb_ref[...] if h == 0 else wqb_comm[h]
        _attn_group(x2d, k_ref, v_ref, cfa_ref, mask, wqa_cur,
                    ja * H_PER, h)
        _attn_group(x2d, k_ref, v_ref, cfb_ref, mask, wqb_cur,
                    jb * H_PER + HC, h)

        if h < N_DEV - 1:
            for rdma in rdmas[h]:
                rdma.wait_recv()

    for hop_rdmas in rdmas:
        for rdma in hop_rdmas:
            rdma.wait_send()

    woa_all = woa_comm[...].reshape(N_DEV * D_HALF, D_MODEL)
    wob_all = wob_comm[...].reshape(N_DEV * D_HALF, D_MODEL)
    out = jnp.dot(cfa_ref[...], woa_all,
                  preferred_element_type=jnp.float32)
    out = out + jnp.dot(cfb_ref[...], wob_all,
                        preferred_element_type=jnp.float32)
    out_ref[...] = out.reshape(B_PER, SQ, D_MODEL)


def kernel(x, Wq, K_ext, V_ext, Wo):
    p = lax.axis_index("i")

    ks = lax.dynamic_slice_in_dim(K_ext, p * B_PER, B_PER, axis=0)
    vs = lax.dynamic_slice_in_dim(V_ext, p * B_PER, B_PER, axis=0)
    kt = jnp.transpose(ks, (0, 2, 3, 1)).astype(jnp.bfloat16)
    vt = jnp.transpose(vs, (0, 2, 3, 1)).astype(jnp.bfloat16)
    xb = x.astype(jnp.bfloat16)
    wq16 = (Wq * 0.125).astype(jnp.bfloat16)
    wo16 = Wo.astype(jnp.bfloat16)
    wqa, wqb = wq16[:, :D_HALF], wq16[:, D_HALF:]
    woa, wob = wo16[:D_HALF], wo16[D_HALF:]

    return pl.pallas_call(
        _body,
        out_shape=jax.ShapeDtypeStruct((B_PER, SQ, D_MODEL), jnp.float32),
        in_specs=[pl.BlockSpec(memory_space=pltpu.VMEM)] * 7,
        out_specs=pl.BlockSpec(memory_space=pltpu.VMEM),
        scratch_shapes=[
            pltpu.VMEM((N_DEV, D_MODEL, D_HALF), jnp.bfloat16),
            pltpu.VMEM((N_DEV, D_HALF, D_MODEL), jnp.bfloat16),
            pltpu.VMEM((N_DEV, D_MODEL, D_HALF), jnp.bfloat16),
            pltpu.VMEM((N_DEV, D_HALF, D_MODEL), jnp.bfloat16),
            pltpu.VMEM((B_PER * SQ, N_DEV * D_HALF), jnp.bfloat16),
            pltpu.VMEM((B_PER * SQ, N_DEV * D_HALF), jnp.bfloat16),
        ] + [pltpu.SemaphoreType.DMA((N_DEV,))] * 8,
        compiler_params=pltpu.CompilerParams(
            collective_id=0, vmem_limit_bytes=100 * 1024 * 1024),
    )(xb, wqa, wqb, woa, wob, kt, vt)


# device time: 125939 ns/iter; 1.5898x vs baseline; 1.0122x over previous
import jax
import jax.numpy as jnp
from jax import lax
from jax.experimental import pallas as pl
from jax.experimental.pallas import tpu as pltpu

N_DEV = 8
B_PER = 2
SQ = 512
SKV = 512
H_PER = 8
HC = 4
DH = 64
D_MODEL = 768
D_HALF = HC * DH
BLK = 64


def _attn_group(x2d, k_ref, v_ref, cf_ref, mask, wq_cur, head0, h):
    qf = jnp.dot(x2d, wq_cur,
                 preferred_element_type=jnp.float32).astype(jnp.bfloat16)
    for b in range(B_PER):
        ctxs = []
        for hl in range(HC):
            q_h = qf[b * SQ:(b + 1) * SQ, hl * DH:(hl + 1) * DH]
            k_h = k_ref[b, pl.ds(head0 + hl, 1)][0]
            v_h = v_ref[b, pl.ds(head0 + hl, 1)][0]
            s = jnp.dot(q_h, k_h,
                        preferred_element_type=jnp.float32)
            e = jnp.where(mask, jnp.exp(s.astype(jnp.bfloat16)),
                          jnp.bfloat16(0.0))
            denom = jnp.sum(e, axis=-1, dtype=jnp.float32)
            ctx = lax.dot_general(
                e, v_h, (((1,), (1,)), ((), ())),
                preferred_element_type=jnp.float32)
            ctxs.append((ctx / denom[:, None]).astype(jnp.bfloat16))
        cf_ref[pl.ds(b * SQ, SQ), pl.ds(h * D_HALF, D_HALF)] = (
            jnp.concatenate(ctxs, axis=1))


_SUCC = (1, 2, 3, 7, 0, 4, 5, 6)
_PRED = (4, 0, 1, 2, 5, 6, 7, 3)


def _perm(table, p):
    out = jnp.int32(table[0])
    for k in range(1, N_DEV):
        out = jnp.where(p == k, jnp.int32(table[k]), out)
    return out


def _body(x_ref, wqa_ref, wqb_ref, woa_ref, wob_ref, k_ref, v_ref, out_ref,
          wqa_comm, woa_comm, wqb_comm, wob_comm, cfa_ref, cfb_ref,
          a_wq_send, a_wq_recv, a_wo_send, a_wo_recv,
          b_wq_send, b_wq_recv, b_wo_send, b_wo_recv):
    my_pos = lax.axis_index("i")
    left = _perm(_PRED, my_pos)
    right = _perm(_SUCC, my_pos)

    barrier_sem = pltpu.get_barrier_semaphore()
    for nbr in (left, right):
        pl.semaphore_signal(barrier_sem, inc=1, device_id=(nbr,),
                            device_id_type=pl.DeviceIdType.MESH)
    pl.semaphore_wait(barrier_sem, 2)

    rows = lax.broadcasted_iota(jnp.int32, (SQ, SKV), 0) // BLK
    cols = lax.broadcasted_iota(jnp.int32, (SQ, SKV), 1) // BLK
    mask = cols <= rows

    x2d = x_ref[...].reshape(B_PER * SQ, D_MODEL)

    woa_comm[0] = woa_ref[...]
    wob_comm[0] = wob_ref[...]

    rdmas = []
    ja = my_pos
    jb = my_pos
    for h in range(N_DEV):
        if h > 0:
            ja = _perm(_PRED, ja)
            jb = _perm(_SUCC, jb)
        if h < N_DEV - 1:
            started = []
            for wq_src0, comm_wq, comm_wo, s_wq, r_wq, s_wo, r_wo, tgt in (
                (wqa_ref, wqa_comm, woa_comm,
                 a_wq_send, a_wq_recv, a_wo_send, a_wo_recv, right),
                (wqb_ref, wqb_comm, wob_comm,
                 b_wq_send, b_wq_recv, b_wo_send, b_wo_recv, left),
            ):
                src_wq = wq_src0 if h == 0 else comm_wq.at[h]
                rdma_wq = pltpu.make_async_remote_copy(
                    src_ref=src_wq, dst_ref=comm_wq.at[h + 1],
                    send_sem=s_wq.at[h], recv_sem=r_wq.at[h + 1],
                    device_id=(tgt,), device_id_type=pl.DeviceIdType.MESH)
                rdma_wo = pltpu.make_async_remote_copy(
                    src_ref=comm_wo.at[h], dst_ref=comm_wo.at[h + 1],
                    send_sem=s_wo.at[h], recv_sem=r_wo.at[h + 1],
                    device_id=(tgt,), device_id_type=pl.DeviceIdType.MESH)
                rdma_wq.start()
                rdma_wo.start()
                started.extend((rdma_wq, rdma_wo))
            rdmas.append(started)

        wqa_cur = wqa_ref[...] if h == 0 else wqa_comm[h]
        wqb_cur = wqb_ref[...] if h == 0 else wqb_comm[h]
        _attn_group(x2d, k_ref, v_ref, cfa_ref, mask, wqa_cur,
                    ja * H_PER, h)
        _attn_group(x2d, k_ref, v_ref, cfb_ref, mask, wqb_cur,
                    jb * H_PER + HC, h)

        if h < N_DEV - 1:
            for rdma in rdmas[h]:
                rdma.wait_recv()

    for hop_rdmas in rdmas:
        for rdma in hop_rdmas:
            rdma.wait_send()

    woa_all = woa_comm[...].reshape(N_DEV * D_HALF, D_MODEL)
    wob_all = wob_comm[...].reshape(N_DEV * D_HALF, D_MODEL)
    out = jnp.dot(cfa_ref[...], woa_all,
                  preferred_element_type=jnp.float32)
    out = out + jnp.dot(cfb_ref[...], wob_all,
                        preferred_element_type=jnp.float32)
    out_ref[...] = out.reshape(B_PER, SQ, D_MODEL)


def kernel(x, Wq, K_ext, V_ext, Wo):
    p = lax.axis_index("i")

    ks = lax.dynamic_slice_in_dim(K_ext, p * B_PER, B_PER, axis=0)
    vs = lax.dynamic_slice_in_dim(V_ext, p * B_PER, B_PER, axis=0)
    kt = jnp.transpose(ks, (0, 2, 3, 1)).astype(jnp.bfloat16)
    vt = jnp.transpose(vs, (0, 2, 3, 1)).astype(jnp.bfloat16)
    xb = x.astype(jnp.bfloat16)
    wq16 = (Wq * 0.125).astype(jnp.bfloat16)
    wo16 = Wo.astype(jnp.bfloat16)
    wqa, wqb = wq16[:, :D_HALF], wq16[:, D_HALF:]
    woa, wob = wo16[:D_HALF], wo16[D_HALF:]

    return pl.pallas_call(
        _body,
        out_shape=jax.ShapeDtypeStruct((B_PER, SQ, D_MODEL), jnp.float32),
        in_specs=[pl.BlockSpec(memory_space=pltpu.VMEM)] * 7,
        out_specs=pl.BlockSpec(memory_space=pltpu.VMEM),
        scratch_shapes=[
            pltpu.VMEM((N_DEV, D_MODEL, D_HALF), jnp.bfloat16),
            pltpu.VMEM((N_DEV, D_HALF, D_MODEL), jnp.bfloat16),
            pltpu.VMEM((N_DEV, D_MODEL, D_HALF), jnp.bfloat16),
            pltpu.VMEM((N_DEV, D_HALF, D_MODEL), jnp.bfloat16),
            pltpu.VMEM((B_PER * SQ, N_DEV * D_HALF), jnp.bfloat16),
            pltpu.VMEM((B_PER * SQ, N_DEV * D_HALF), jnp.bfloat16),
        ] + [pltpu.SemaphoreType.DMA((N_DEV,))] * 8,
        compiler_params=pltpu.CompilerParams(
            collective_id=0, vmem_limit_bytes=100 * 1024 * 1024),
    )(xb, wqa, wqb, woa, wob, kt, vt)
